# baseline (speedup 1.0000x reference)
import functools

import jax
import jax.numpy as jnp
from jax import lax
from jax.experimental import pallas as pl
from jax.experimental.pallas import tpu as pltpu
from jax.experimental.pallas import tpu_sc as plsc

_N = 10000
_E = 160000
_DIN = 256
_H = 128
_DOUT = 256

_NW = 32
_C = 128
_NCH = 40
_EPW = _C * _NCH
_EPAD = _NW * _EPW - _E
_NP = 10240
_RPT = _NP // 16
_CR = _NP // 128


def _make_count():
  mesh = plsc.VectorSubcoreMesh(core_axis_name="c", subcore_axis_name="s")

  @functools.partial(
      pl.kernel,
      out_type=[
          jax.ShapeDtypeStruct((_NP, 8), jnp.float32),
          jax.ShapeDtypeStruct((_NP, 8), jnp.float32),
      ],
      mesh=mesh,
      compiler_params=pltpu.CompilerParams(needs_layout_passes=False),
      scratch_types=[
          pltpu.VMEM((_NCH, _C), jnp.int32),
          pltpu.VMEM((_CR, 128), jnp.float32),
          pltpu.VMEM((_CR,), jnp.int32),
          pltpu.VMEM_SHARED((_CR, 128), jnp.float32),
          pltpu.VMEM((5, 128), jnp.float32),
          pltpu.VMEM((_RPT, 8), jnp.float32),
      ],
  )
  def count(dsts, zrows, outc0, outc1, dst_v, cnt_v, iota_v, cacc, strip_v, col_v):
    c = lax.axis_index("c")
    s = lax.axis_index("s")
    wid = c * 16 + s
    pltpu.sync_copy(zrows.at[pl.ds(0, _CR)], cnt_v)
    for k in range(_CR // 16):
      iota_v[pl.ds(16 * k, 16)] = lax.iota(jnp.int32, 16) + 16 * k

    @pl.when(s == 0)
    def _():
      pltpu.sync_copy(zrows.at[pl.ds(0, _CR)], cacc)

    pltpu.sync_copy(dsts.at[wid], dst_v)
    plsc.subcore_barrier()

    ones16 = jnp.full((16,), 1.0, jnp.float32)

    def cbody(j, carry):
      for k in range(_C // 16):
        d16 = dst_v[j, pl.ds(k * 16, 16)]
        plsc.addupdate_scatter(
            cnt_v, [lax.shift_right_logical(d16, 7),
                    lax.bitwise_and(d16, 127)], ones16)
      return carry

    lax.fori_loop(0, _NCH, cbody, 0)
    pltpu.sync_copy(cnt_v, cacc.at[iota_v], add=True)
    plsc.subcore_barrier()
    pltpu.sync_copy(cacc.at[pl.ds(s * 5, 5)], strip_v)

    def ebody(i, carry):
      v = strip_v[i // 8, pl.ds((i % 8) * 16, 16)]
      n16 = lax.iota(jnp.int32, 16) + 16 * i
      for k in range(8):
        plsc.store_scatter(col_v, [n16, jnp.full((16,), k, jnp.int32)], v)
      return carry

    lax.fori_loop(0, _RPT // 16, ebody, 0)

    @pl.when(c == 0)
    def _():
      pltpu.sync_copy(col_v, outc0.at[pl.ds(s * _RPT, _RPT)])

    @pl.when(c == 1)
    def _():
      pltpu.sync_copy(col_v, outc1.at[pl.ds(s * _RPT, _RPT)])

  return count


def _make_segsum():
  mesh = plsc.VectorSubcoreMesh(core_axis_name="c", subcore_axis_name="s")

  @functools.partial(
      pl.kernel,
      out_type=[
          jax.ShapeDtypeStruct((_NP, _H), jnp.float32),
          jax.ShapeDtypeStruct((_NP, _H), jnp.float32),
      ],
      mesh=mesh,
      compiler_params=pltpu.CompilerParams(needs_layout_passes=False),
      scratch_types=[
          pltpu.VMEM((_NCH, _C), jnp.int32),
          pltpu.VMEM((_NCH, _C), jnp.int32),
          pltpu.VMEM((_C, _H), jnp.float32),
          pltpu.VMEM((_C, _H), jnp.float32),
          pltpu.VMEM_SHARED((_NP, _H), jnp.float32),
          pltpu.SemaphoreType.DMA,
          pltpu.SemaphoreType.DMA,
          pltpu.SemaphoreType.DMA,
          pltpu.SemaphoreType.DMA,
      ],
  )
  def segsum(table, srcs, dsts, zrows, out0, out1, src_v, dst_v, rows0_v,
             rows1_v, acc, gsem0, gsem1, ssem0, ssem1):
    c = lax.axis_index("c")
    s = lax.axis_index("s")
    wid = c * 16 + s
    pltpu.sync_copy(zrows, acc.at[pl.ds(s * _RPT, _RPT)])
    pltpu.sync_copy(srcs.at[wid], src_v)
    pltpu.sync_copy(dsts.at[wid], dst_v)
    plsc.subcore_barrier()

    pltpu.async_copy(table.at[src_v.at[0]], rows0_v, gsem0)

    def body(k, carry):
      j0 = 2 * k
      pltpu.async_copy(table.at[src_v.at[j0 + 1]], rows1_v, gsem1)
      pltpu.make_async_copy(table.at[src_v.at[j0]], rows0_v, gsem0).wait()
      pltpu.sync_copy(rows0_v, acc.at[dst_v.at[j0]], add=True)

      @pl.when(k < _NCH // 2 - 1)
      def _():
        pltpu.async_copy(table.at[src_v.at[j0 + 2]], rows0_v, gsem0)

      pltpu.make_async_copy(table.at[src_v.at[j0 + 1]], rows1_v, gsem1).wait()
      pltpu.sync_copy(rows1_v, acc.at[dst_v.at[j0 + 1]], add=True)
      return carry

    lax.fori_loop(0, _NCH // 2, body, 0)
    plsc.subcore_barrier()

    @pl.when(c == 0)
    def _():
      pltpu.sync_copy(acc.at[pl.ds(s * _RPT, _RPT)],
                      out0.at[pl.ds(s * _RPT, _RPT)])

    @pl.when(c == 1)
    def _():
      pltpu.sync_copy(acc.at[pl.ds(s * _RPT, _RPT)],
                      out1.at[pl.ds(s * _RPT, _RPT)])

  return segsum


_count = _make_count()
_segsum = _make_segsum()

_BR = 2000


def _pre_body(x_ref, w1lt_ref, w1rt_ref, p1_ref, r1_ref):
  p1_ref[:] = jnp.dot(x_ref[:], w1lt_ref[:], preferred_element_type=jnp.float32)
  r1_ref[:] = jnp.dot(x_ref[:], w1rt_ref[:], preferred_element_type=jnp.float32)


def _mid_body(p0_ref, p1_ref, c0_ref, c1_ref, r1_ref, b1l_ref, h_ref, inv_ref):
  cnt = c0_ref[:, :1] + c1_ref[:, :1]
  inv = 1.0 / jnp.maximum(cnt, 1.0)
  s = p0_ref[:] + p1_ref[:]
  h_ref[:] = jnp.maximum(s * inv + b1l_ref[:] + r1_ref[:], 0.0)
  inv_ref[:] = jnp.broadcast_to(inv, (_BR, _H))


def _post_body(p0_ref, p1_ref, inv_ref, h_ref, w2lt_ref, b2l_ref, w2rt_ref,
               z_ref):
  mean2 = (p0_ref[:] + p1_ref[:]) * inv_ref[:]
  z_ref[:] = (jnp.dot(mean2, w2lt_ref[:], preferred_element_type=jnp.float32)
              + b2l_ref[:]
              + jnp.dot(h_ref[:], w2rt_ref[:],
                        preferred_element_type=jnp.float32))


def _row_spec(d):
  return pl.BlockSpec((_BR, d), lambda i: (i, 0))


def _full_spec(r, d):
  return pl.BlockSpec((r, d), lambda i: (0, 0))


_pre = pl.pallas_call(
    _pre_body,
    grid=(_N // _BR,),
    in_specs=[_row_spec(_DIN), _full_spec(_DIN, _H), _full_spec(_DIN, _H)],
    out_specs=[_row_spec(_H), _row_spec(_H)],
    out_shape=[
        jax.ShapeDtypeStruct((_N, _H), jnp.float32),
        jax.ShapeDtypeStruct((_N, _H), jnp.float32),
    ],
)

_mid = pl.pallas_call(
    _mid_body,
    grid=(_N // _BR,),
    in_specs=[_row_spec(_H), _row_spec(_H),
              pl.BlockSpec((_BR, 8), lambda i: (i, 0)),
              pl.BlockSpec((_BR, 8), lambda i: (i, 0)),
              _row_spec(_H), _full_spec(1, _H)],
    out_specs=[_row_spec(_H), _row_spec(_H)],
    out_shape=[
        jax.ShapeDtypeStruct((_N, _H), jnp.float32),
        jax.ShapeDtypeStruct((_N, _H), jnp.float32),
    ],
)

_post = pl.pallas_call(
    _post_body,
    grid=(_N // _BR,),
    in_specs=[_row_spec(_H), _row_spec(_H), _row_spec(_H), _row_spec(_H),
              _full_spec(_H, _DOUT), _full_spec(1, _DOUT),
              _full_spec(_H, _DOUT)],
    out_specs=_row_spec(_DOUT),
    out_shape=jax.ShapeDtypeStruct((_N, _DOUT), jnp.float32),
)


@jax.jit
def kernel(x, edge_index, W1l, b1l, W1r, W2l, b2l, W2r):
  ppw = _EPAD // _NW
  wi = lax.broadcasted_iota(jnp.int32, (_NW, ppw), 0)
  pi = lax.broadcasted_iota(jnp.int32, (_NW, ppw), 1)
  pad_src = (wi * ppw + pi) % _N
  pad_dst = _N + (wi * ppw + pi) % (_NP - _N)
  srcs = jnp.concatenate(
      [edge_index[0].reshape(_NW, _E // _NW), pad_src], axis=1
  ).reshape(_NW, _NCH, _C)
  dsts = jnp.concatenate(
      [edge_index[1].reshape(_NW, _E // _NW), pad_dst], axis=1
  ).reshape(_NW, _NCH, _C)
  zrows = jnp.zeros((_RPT, _H), jnp.float32)

  c0v, c1v = _count(dsts, zrows)
  p1, r1 = _pre(x, W1l.T, W1r.T)
  p1a, p1b = _segsum(p1, srcs, dsts, zrows)
  h, inv = _mid(p1a, p1b, c0v, c1v, r1, b1l.reshape(1, _H))
  p2a, p2b = _segsum(h, srcs, dsts, zrows)
  z = _post(p2a, p2b, inv, h, W2l.T, b2l.reshape(1, _DOUT), W2r.T)
  return z


# speedup vs baseline: 1.0409x; 1.0409x over previous
import functools

import jax
import jax.numpy as jnp
from jax import lax
from jax.experimental import pallas as pl
from jax.experimental.pallas import tpu as pltpu
from jax.experimental.pallas import tpu_sc as plsc

_N = 10000
_E = 160000
_DIN = 256
_H = 128
_DOUT = 256

_NW = 32
_C = 128
_NCH = 40
_EPW = _C * _NCH
_EPAD = _NW * _EPW - _E
_NP = 10240
_RPT = _NP // 16
_CR = _NP // 128


def _make_count():
  mesh = plsc.VectorSubcoreMesh(core_axis_name="c", subcore_axis_name="s")

  @functools.partial(
      pl.kernel,
      out_type=jax.ShapeDtypeStruct((2, _NP, 8), jnp.float32),
      mesh=mesh,
      compiler_params=pltpu.CompilerParams(needs_layout_passes=False),
      scratch_types=[
          pltpu.VMEM((_NCH, _C), jnp.int32),
          pltpu.VMEM((_CR, 128), jnp.float32),
          pltpu.VMEM((_CR,), jnp.int32),
          pltpu.VMEM_SHARED((_CR, 128), jnp.float32),
          pltpu.VMEM((5, 128), jnp.float32),
          pltpu.VMEM((_RPT, 8), jnp.float32),
      ],
  )
  def count(dsts, zrows, outc, dst_v, cnt_v, iota_v, cacc, strip_v, col_v):
    c = lax.axis_index("c")
    s = lax.axis_index("s")
    wid = c * 16 + s
    pltpu.sync_copy(zrows.at[pl.ds(0, _CR)], cnt_v)
    for k in range(_CR // 16):
      iota_v[pl.ds(16 * k, 16)] = lax.iota(jnp.int32, 16) + 16 * k

    @pl.when(s == 0)
    def _():
      pltpu.sync_copy(zrows.at[pl.ds(0, _CR)], cacc)

    pltpu.sync_copy(dsts.at[wid], dst_v)
    plsc.subcore_barrier()

    ones16 = jnp.full((16,), 1.0, jnp.float32)

    def cbody(j, carry):
      for k in range(_C // 16):
        d16 = dst_v[j, pl.ds(k * 16, 16)]
        plsc.addupdate_scatter(
            cnt_v, [lax.shift_right_logical(d16, 7),
                    lax.bitwise_and(d16, 127)], ones16)
      return carry

    lax.fori_loop(0, _NCH, cbody, 0)
    pltpu.sync_copy(cnt_v, cacc.at[iota_v], add=True)
    plsc.subcore_barrier()
    pltpu.sync_copy(cacc.at[pl.ds(s * 5, 5)], strip_v)

    def ebody(i, carry):
      v = strip_v[i // 8, pl.ds((i % 8) * 16, 16)]
      n16 = lax.iota(jnp.int32, 16) + 16 * i
      for k in range(8):
        plsc.store_scatter(col_v, [n16, jnp.full((16,), k, jnp.int32)], v)
      return carry

    lax.fori_loop(0, _RPT // 16, ebody, 0)
    pltpu.sync_copy(col_v, outc.at[c, pl.ds(s * _RPT, _RPT)])

  return count


def _make_segsum():
  mesh = plsc.VectorSubcoreMesh(core_axis_name="c", subcore_axis_name="s")

  @functools.partial(
      pl.kernel,
      out_type=[
          jax.ShapeDtypeStruct((_NP, _H), jnp.float32),
          jax.ShapeDtypeStruct((_NP, _H), jnp.float32),
      ],
      mesh=mesh,
      compiler_params=pltpu.CompilerParams(needs_layout_passes=False),
      scratch_types=[
          pltpu.VMEM((_NCH, _C), jnp.int32),
          pltpu.VMEM((_NCH, _C), jnp.int32),
          pltpu.VMEM((_C, _H), jnp.float32),
          pltpu.VMEM((_C, _H), jnp.float32),
          pltpu.VMEM_SHARED((_NP, _H), jnp.float32),
          pltpu.SemaphoreType.DMA,
          pltpu.SemaphoreType.DMA,
          pltpu.SemaphoreType.DMA,
          pltpu.SemaphoreType.DMA,
      ],
  )
  def segsum(table, srcs, dsts, zrows, out0, out1, src_v, dst_v, rows0_v,
             rows1_v, acc, gsem0, gsem1, ssem0, ssem1):
    c = lax.axis_index("c")
    s = lax.axis_index("s")
    wid = c * 16 + s
    pltpu.sync_copy(zrows, acc.at[pl.ds(s * _RPT, _RPT)])
    pltpu.sync_copy(srcs.at[wid], src_v)
    pltpu.sync_copy(dsts.at[wid], dst_v)
    plsc.subcore_barrier()

    pltpu.async_copy(table.at[src_v.at[0]], rows0_v, gsem0)

    def body(k, carry):
      j0 = 2 * k
      pltpu.async_copy(table.at[src_v.at[j0 + 1]], rows1_v, gsem1)
      pltpu.make_async_copy(table.at[src_v.at[j0]], rows0_v, gsem0).wait()
      pltpu.sync_copy(rows0_v, acc.at[dst_v.at[j0]], add=True)

      @pl.when(k < _NCH // 2 - 1)
      def _():
        pltpu.async_copy(table.at[src_v.at[j0 + 2]], rows0_v, gsem0)

      pltpu.make_async_copy(table.at[src_v.at[j0 + 1]], rows1_v, gsem1).wait()
      pltpu.sync_copy(rows1_v, acc.at[dst_v.at[j0 + 1]], add=True)
      return carry

    lax.fori_loop(0, _NCH // 2, body, 0)
    plsc.subcore_barrier()

    @pl.when(c == 0)
    def _():
      pltpu.sync_copy(acc.at[pl.ds(s * _RPT, _RPT)],
                      out0.at[pl.ds(s * _RPT, _RPT)])

    @pl.when(c == 1)
    def _():
      pltpu.sync_copy(acc.at[pl.ds(s * _RPT, _RPT)],
                      out1.at[pl.ds(s * _RPT, _RPT)])

  return segsum


_count = _make_count()
_segsum = _make_segsum()

_BR = 2000


def _pre_body(x_ref, w1lt_ref, w1rt_ref, p1_ref, r1_ref):
  p1_ref[:] = jnp.dot(x_ref[:], w1lt_ref[:], preferred_element_type=jnp.float32)
  r1_ref[:] = jnp.dot(x_ref[:], w1rt_ref[:], preferred_element_type=jnp.float32)


def _mid_body(p0_ref, p1_ref, c0_ref, c1_ref, r1_ref, b1l_ref, h_ref, inv_ref):
  cnt = c0_ref[:, :1] + c1_ref[:, :1]
  inv = 1.0 / jnp.maximum(cnt, 1.0)
  s = p0_ref[:] + p1_ref[:]
  h_ref[:] = jnp.maximum(s * inv + b1l_ref[:] + r1_ref[:], 0.0)
  inv_ref[:] = jnp.broadcast_to(inv, (_BR, _H))


def _post_body(p0_ref, p1_ref, inv_ref, h_ref, w2lt_ref, b2l_ref, w2rt_ref,
               z_ref):
  mean2 = (p0_ref[:] + p1_ref[:]) * inv_ref[:]
  z_ref[:] = (jnp.dot(mean2, w2lt_ref[:], preferred_element_type=jnp.float32)
              + b2l_ref[:]
              + jnp.dot(h_ref[:], w2rt_ref[:],
                        preferred_element_type=jnp.float32))


def _row_spec(d):
  return pl.BlockSpec((_BR, d), lambda i: (i, 0))


def _full_spec(r, d):
  return pl.BlockSpec((r, d), lambda i: (0, 0))


_pre = pl.pallas_call(
    _pre_body,
    grid=(_N // _BR,),
    in_specs=[_row_spec(_DIN), _full_spec(_DIN, _H), _full_spec(_DIN, _H)],
    out_specs=[_row_spec(_H), _row_spec(_H)],
    out_shape=[
        jax.ShapeDtypeStruct((_N, _H), jnp.float32),
        jax.ShapeDtypeStruct((_N, _H), jnp.float32),
    ],
)

_mid = pl.pallas_call(
    _mid_body,
    grid=(_N // _BR,),
    in_specs=[_row_spec(_H), _row_spec(_H),
              pl.BlockSpec((_BR, 8), lambda i: (i, 0)),
              pl.BlockSpec((_BR, 8), lambda i: (i, 0)),
              _row_spec(_H), _full_spec(1, _H)],
    out_specs=[_row_spec(_H), _row_spec(_H)],
    out_shape=[
        jax.ShapeDtypeStruct((_N, _H), jnp.float32),
        jax.ShapeDtypeStruct((_N, _H), jnp.float32),
    ],
)

_post = pl.pallas_call(
    _post_body,
    grid=(_N // _BR,),
    in_specs=[_row_spec(_H), _row_spec(_H), _row_spec(_H), _row_spec(_H),
              _full_spec(_H, _DOUT), _full_spec(1, _DOUT),
              _full_spec(_H, _DOUT)],
    out_specs=_row_spec(_DOUT),
    out_shape=jax.ShapeDtypeStruct((_N, _DOUT), jnp.float32),
)


@jax.jit
def kernel(x, edge_index, W1l, b1l, W1r, W2l, b2l, W2r):
  ppw = _EPAD // _NW
  wi = lax.broadcasted_iota(jnp.int32, (_NW, ppw), 0)
  pi = lax.broadcasted_iota(jnp.int32, (_NW, ppw), 1)
  pad_src = (wi * ppw + pi) % _N
  pad_dst = _N + (wi * ppw + pi) % (_NP - _N)
  srcs = jnp.concatenate(
      [edge_index[0].reshape(_NW, _E // _NW), pad_src], axis=1
  ).reshape(_NW, _NCH, _C)
  dsts = jnp.concatenate(
      [edge_index[1].reshape(_NW, _E // _NW), pad_dst], axis=1
  ).reshape(_NW, _NCH, _C)
  zrows = jnp.zeros((_RPT, _H), jnp.float32)

  cnts = _count(dsts, zrows)
  p1, r1 = _pre(x, W1l.T, W1r.T)
  p1a, p1b = _segsum(p1, srcs, dsts, zrows)
  h, inv = _mid(p1a, p1b, cnts[0], cnts[1], r1, b1l.reshape(1, _H))
  p2a, p2b = _segsum(h, srcs, dsts, zrows)
  z = _post(p2a, p2b, inv, h, W2l.T, b2l.reshape(1, _DOUT), W2r.T)
  return z


# speedup vs baseline: 1.0420x; 1.0011x over previous
import functools

import jax
import jax.numpy as jnp
from jax import lax
from jax.experimental import pallas as pl
from jax.experimental.pallas import tpu as pltpu
from jax.experimental.pallas import tpu_sc as plsc

_N = 10000
_E = 160000
_DIN = 256
_H = 128
_DOUT = 256

_NW = 32
_C = 128
_NCH = 40
_EPW = _C * _NCH
_EPAD = _NW * _EPW - _E
_NP = 10240
_RPT = _NP // 16
_CR = _NP // 128


def _make_count():
  mesh = plsc.VectorSubcoreMesh(core_axis_name="c", subcore_axis_name="s")

  @functools.partial(
      pl.kernel,
      out_type=jax.ShapeDtypeStruct((2, _NP, 8), jnp.float32),
      mesh=mesh,
      compiler_params=pltpu.CompilerParams(needs_layout_passes=False),
      scratch_types=[
          pltpu.VMEM((_NCH, _C), jnp.int32),
          pltpu.VMEM((_CR, 128), jnp.float32),
          pltpu.VMEM((_CR,), jnp.int32),
          pltpu.VMEM_SHARED((_CR, 128), jnp.float32),
          pltpu.VMEM((5, 128), jnp.float32),
          pltpu.VMEM((_RPT, 8), jnp.float32),
      ],
  )
  def count(dsts, zrows, outc, dst_v, cnt_v, iota_v, cacc, strip_v, col_v):
    c = lax.axis_index("c")
    s = lax.axis_index("s")
    wid = c * 16 + s
    pltpu.sync_copy(zrows.at[pl.ds(0, _CR)], cnt_v)
    for k in range(_CR // 16):
      iota_v[pl.ds(16 * k, 16)] = lax.iota(jnp.int32, 16) + 16 * k

    @pl.when(s == 0)
    def _():
      pltpu.sync_copy(zrows.at[pl.ds(0, _CR)], cacc)

    pltpu.sync_copy(dsts.at[wid], dst_v)
    plsc.subcore_barrier()

    ones16 = jnp.full((16,), 1.0, jnp.float32)

    def cbody(j, carry):
      for k in range(_C // 16):
        d16 = dst_v[j, pl.ds(k * 16, 16)]
        plsc.addupdate_scatter(
            cnt_v, [lax.shift_right_logical(d16, 7),
                    lax.bitwise_and(d16, 127)], ones16)
      return carry

    lax.fori_loop(0, _NCH, cbody, 0)
    pltpu.sync_copy(cnt_v, cacc.at[iota_v], add=True)
    plsc.subcore_barrier()
    pltpu.sync_copy(cacc.at[pl.ds(s * 5, 5)], strip_v)

    def ebody(i, carry):
      v = strip_v[i // 8, pl.ds((i % 8) * 16, 16)]
      n16 = lax.iota(jnp.int32, 16) + 16 * i
      for k in range(8):
        plsc.store_scatter(col_v, [n16, jnp.full((16,), k, jnp.int32)], v)
      return carry

    lax.fori_loop(0, _RPT // 16, ebody, 0)
    pltpu.sync_copy(col_v, outc.at[c, pl.ds(s * _RPT, _RPT)])

  return count


def _make_segsum():
  mesh = plsc.VectorSubcoreMesh(core_axis_name="c", subcore_axis_name="s")

  @functools.partial(
      pl.kernel,
      out_type=[
          jax.ShapeDtypeStruct((_NP, _H), jnp.float32),
          jax.ShapeDtypeStruct((_NP, _H), jnp.float32),
      ],
      mesh=mesh,
      compiler_params=pltpu.CompilerParams(needs_layout_passes=False),
      scratch_types=[
          pltpu.VMEM((_NCH, _C), jnp.int32),
          pltpu.VMEM((_NCH, _C), jnp.int32),
          pltpu.VMEM((_C, _H), jnp.float32),
          pltpu.VMEM((_C, _H), jnp.float32),
          pltpu.VMEM_SHARED((_NP, _H), jnp.float32),
          pltpu.SemaphoreType.DMA,
          pltpu.SemaphoreType.DMA,
      ],
  )
  def segsum(table, srcs, dsts, zrows, out0, out1, src_v, dst_v, rows0_v,
             rows1_v, acc, gsem0, gsem1):
    c = lax.axis_index("c")
    s = lax.axis_index("s")
    wid = c * 16 + s
    pltpu.sync_copy(zrows, acc.at[pl.ds(s * _RPT, _RPT)])
    pltpu.sync_copy(srcs.at[wid], src_v)
    pltpu.sync_copy(dsts.at[wid], dst_v)
    plsc.subcore_barrier()

    pltpu.async_copy(table.at[src_v.at[0]], rows0_v, gsem0)

    def body(k, carry):
      j0 = 2 * k
      pltpu.async_copy(table.at[src_v.at[j0 + 1]], rows1_v, gsem1)
      pltpu.make_async_copy(table.at[src_v.at[j0]], rows0_v, gsem0).wait()
      pltpu.sync_copy(rows0_v, acc.at[dst_v.at[j0]], add=True)

      @pl.when(k < _NCH // 2 - 1)
      def _():
        pltpu.async_copy(table.at[src_v.at[j0 + 2]], rows0_v, gsem0)

      pltpu.make_async_copy(table.at[src_v.at[j0 + 1]], rows1_v, gsem1).wait()
      pltpu.sync_copy(rows1_v, acc.at[dst_v.at[j0 + 1]], add=True)
      return carry

    lax.fori_loop(0, _NCH // 2, body, 0)
    plsc.subcore_barrier()

    @pl.when(c == 0)
    def _():
      pltpu.sync_copy(acc.at[pl.ds(s * _RPT, _RPT)],
                      out0.at[pl.ds(s * _RPT, _RPT)])

    @pl.when(c == 1)
    def _():
      pltpu.sync_copy(acc.at[pl.ds(s * _RPT, _RPT)],
                      out1.at[pl.ds(s * _RPT, _RPT)])

  return segsum


_count = _make_count()
_segsum = _make_segsum()

_BR = 2000


def _pre_body(x_ref, w1lt_ref, w1rt_ref, p1_ref, r1_ref):
  p1_ref[:] = jnp.dot(x_ref[:], w1lt_ref[:], preferred_element_type=jnp.float32)
  r1_ref[:] = jnp.dot(x_ref[:], w1rt_ref[:], preferred_element_type=jnp.float32)


def _mid_body(p0_ref, p1_ref, c0_ref, c1_ref, r1_ref, b1l_ref, h_ref, inv_ref):
  cnt = c0_ref[:, :1] + c1_ref[:, :1]
  inv = 1.0 / jnp.maximum(cnt, 1.0)
  s = p0_ref[:] + p1_ref[:]
  h_ref[:] = jnp.maximum(s * inv + b1l_ref[:] + r1_ref[:], 0.0)
  inv_ref[:] = jnp.broadcast_to(inv, (_BR, _H))


def _post_body(p0_ref, p1_ref, inv_ref, h_ref, w2lt_ref, b2l_ref, w2rt_ref,
               z_ref):
  mean2 = (p0_ref[:] + p1_ref[:]) * inv_ref[:]
  z_ref[:] = (jnp.dot(mean2, w2lt_ref[:], preferred_element_type=jnp.float32)
              + b2l_ref[:]
              + jnp.dot(h_ref[:], w2rt_ref[:],
                        preferred_element_type=jnp.float32))


def _row_spec(d):
  return pl.BlockSpec((_BR, d), lambda i: (i, 0))


def _full_spec(r, d):
  return pl.BlockSpec((r, d), lambda i: (0, 0))


_pre = pl.pallas_call(
    _pre_body,
    grid=(_N // _BR,),
    in_specs=[_row_spec(_DIN), _full_spec(_DIN, _H), _full_spec(_DIN, _H)],
    out_specs=[_row_spec(_H), _row_spec(_H)],
    out_shape=[
        jax.ShapeDtypeStruct((_N, _H), jnp.float32),
        jax.ShapeDtypeStruct((_N, _H), jnp.float32),
    ],
)

_mid = pl.pallas_call(
    _mid_body,
    grid=(_N // _BR,),
    in_specs=[_row_spec(_H), _row_spec(_H),
              pl.BlockSpec((_BR, 8), lambda i: (i, 0)),
              pl.BlockSpec((_BR, 8), lambda i: (i, 0)),
              _row_spec(_H), _full_spec(1, _H)],
    out_specs=[_row_spec(_H), _row_spec(_H)],
    out_shape=[
        jax.ShapeDtypeStruct((_N, _H), jnp.float32),
        jax.ShapeDtypeStruct((_N, _H), jnp.float32),
    ],
)

_post = pl.pallas_call(
    _post_body,
    grid=(_N // _BR,),
    in_specs=[_row_spec(_H), _row_spec(_H), _row_spec(_H), _row_spec(_H),
              _full_spec(_H, _DOUT), _full_spec(1, _DOUT),
              _full_spec(_H, _DOUT)],
    out_specs=_row_spec(_DOUT),
    out_shape=jax.ShapeDtypeStruct((_N, _DOUT), jnp.float32),
)


@jax.jit
def kernel(x, edge_index, W1l, b1l, W1r, W2l, b2l, W2r):
  ppw = _EPAD // _NW
  wi = lax.broadcasted_iota(jnp.int32, (_NW, ppw), 0)
  pi = lax.broadcasted_iota(jnp.int32, (_NW, ppw), 1)
  pad_src = (wi * ppw + pi) % _N
  pad_dst = _N + (wi * ppw + pi) % (_NP - _N)
  srcs = jnp.concatenate(
      [edge_index[0].reshape(_NW, _E // _NW), pad_src], axis=1
  ).reshape(_NW, _NCH, _C)
  dsts = jnp.concatenate(
      [edge_index[1].reshape(_NW, _E // _NW), pad_dst], axis=1
  ).reshape(_NW, _NCH, _C)
  zrows = jnp.zeros((_RPT, _H), jnp.float32)

  cnts = _count(dsts, zrows)
  p1, r1 = _pre(x, W1l.T, W1r.T)
  p1a, p1b = _segsum(p1, srcs, dsts, zrows)
  h, inv = _mid(p1a, p1b, cnts[0], cnts[1], r1, b1l.reshape(1, _H))
  p2a, p2b = _segsum(h, srcs, dsts, zrows)
  z = _post(p2a, p2b, inv, h, W2l.T, b2l.reshape(1, _DOUT), W2r.T)
  return z
